# Initial kernel scaffold; baseline (speedup 1.0000x reference)
import jax
import jax.numpy as jnp
from jax.experimental import pallas as pl


def kernel(x, edge_index, att1_Wsrc, att1_bsrc, att1_Wdst, att1_bdst, att1_attn, att1_bias, att1_resW, def1_Wsrc, def1_bsrc, def1_Wdst, def1_bdst, def1_attn, def1_bias, def1_resW, def2_Wsrc, def2_bsrc, def2_Wdst, def2_bdst, def2_attn, def2_bias, W1, b1, W2, b2, W3, b3, W4, b4):
    raise NotImplementedError("write your pallas kernel here")


# speedup vs baseline: 1.0588x; 1.0588x over previous
import functools

import jax
import jax.numpy as jnp
from jax.experimental import pallas as pl
from jax.experimental.pallas import tpu as pltpu

N = 10000
D = 16
H = 2
IN_F = 5 * D
HID = IN_F ** 2

BM = 1000


def _leaky(v, slope):
    return jnp.where(v > 0, v, slope * v)


def _mlp1_body(z_ref, w_ref, b_ref, o_ref):
    acc = jnp.dot(z_ref[...], w_ref[...], preferred_element_type=jnp.float32)
    o_ref[...] = _leaky(acc + b_ref[...], 0.01)


def _mlp1(z, W1, b1):
    return pl.pallas_call(
        _mlp1_body,
        grid=(N // BM,),
        in_specs=[
            pl.BlockSpec((BM, IN_F), lambda m: (m, 0)),
            pl.BlockSpec((IN_F, HID), lambda m: (0, 0)),
            pl.BlockSpec((1, HID), lambda m: (0, 0)),
        ],
        out_specs=pl.BlockSpec((BM, HID), lambda m: (m, 0)),
        out_shape=jax.ShapeDtypeStruct((N, HID), jnp.float32),
    )(z, W1, b1.reshape(1, HID))


def _mlp2_body(h_ref, w_ref, b_ref, o_ref, acc_ref, *, nk):
    k = pl.program_id(2)

    @pl.when(k == 0)
    def _():
        acc_ref[...] = jnp.zeros_like(acc_ref)

    acc_ref[...] += jnp.dot(h_ref[...], w_ref[...],
                            preferred_element_type=jnp.float32)

    @pl.when(k == nk - 1)
    def _():
        o_ref[...] = _leaky(acc_ref[...] + b_ref[...], 0.01)


def _mlp2(h1, W2, b2):
    bn, bk = 1280, 640
    nk = HID // bk
    return pl.pallas_call(
        functools.partial(_mlp2_body, nk=nk),
        grid=(N // BM, HID // bn, nk),
        in_specs=[
            pl.BlockSpec((BM, bk), lambda m, n, k: (m, k)),
            pl.BlockSpec((bk, bn), lambda m, n, k: (k, n)),
            pl.BlockSpec((1, bn), lambda m, n, k: (0, n)),
        ],
        out_specs=pl.BlockSpec((BM, bn), lambda m, n, k: (m, n)),
        out_shape=jax.ShapeDtypeStruct((N, HID), jnp.float32),
        scratch_shapes=[pltpu.VMEM((BM, bn), jnp.float32)],
    )(h1, W2, b2.reshape(1, HID))


def _mlp34_body(h_ref, w3_ref, b3_ref, w4_ref, b4_ref, o_ref):
    h3 = jnp.dot(h_ref[...], w3_ref[...], preferred_element_type=jnp.float32)
    h3 = _leaky(h3 + b3_ref[...], 0.01)
    h4 = jnp.dot(h3, w4_ref[...], preferred_element_type=jnp.float32)
    o_ref[...] = jax.nn.sigmoid(h4 + b4_ref[...])


def _mlp34(h2, W3, b3, W4, b4):
    return pl.pallas_call(
        _mlp34_body,
        grid=(N // BM,),
        in_specs=[
            pl.BlockSpec((BM, HID), lambda m: (m, 0)),
            pl.BlockSpec((HID, IN_F), lambda m: (0, 0)),
            pl.BlockSpec((1, IN_F), lambda m: (0, 0)),
            pl.BlockSpec((IN_F, 1), lambda m: (0, 0)),
            pl.BlockSpec((1, 1), lambda m: (0, 0)),
        ],
        out_specs=pl.BlockSpec((BM, 1), lambda m: (m, 0)),
        out_shape=jax.ShapeDtypeStruct((N, 1), jnp.float32),
    )(h2, W3, b3.reshape(1, IN_F), W4, b4.reshape(1, 1))


def _gat_layer(x, src, dst, Wsrc, bsrc, Wdst, bdst, attn, bias, resW):
    fsrc = (x @ Wsrc + bsrc).reshape(N, H, D)
    fdst = (x @ Wdst + bdst).reshape(N, H, D)
    e = jnp.sum(jax.nn.leaky_relu(fsrc[src] + fdst[dst], 0.2) * attn[None],
                axis=-1)
    ex = jnp.exp(e)
    denom = jax.ops.segment_sum(ex, dst, num_segments=N)
    num = jax.ops.segment_sum(ex[:, :, None] * fsrc[src], dst, num_segments=N)
    rst = num / jnp.maximum(denom, 1e-16)[:, :, None]
    resval = x.reshape(N, H, D) if resW is None else (x @ resW).reshape(N, H, D)
    rst = rst + resval + bias.reshape(1, H, D)
    return jax.nn.elu(rst).reshape(N, H * D)


def kernel(x, edge_index,
           att1_Wsrc, att1_bsrc, att1_Wdst, att1_bdst, att1_attn, att1_bias, att1_resW,
           def1_Wsrc, def1_bsrc, def1_Wdst, def1_bdst, def1_attn, def1_bias, def1_resW,
           def2_Wsrc, def2_bsrc, def2_Wdst, def2_bdst, def2_attn, def2_bias,
           W1, b1, W2, b2, W3, b3, W4, b4):
    src = edge_index[0]
    dst = edge_index[1]
    h_att1 = _gat_layer(x, src, dst, att1_Wsrc, att1_bsrc, att1_Wdst,
                        att1_bdst, att1_attn, att1_bias, att1_resW)
    h_def1 = _gat_layer(x, src, dst, def1_Wsrc, def1_bsrc, def1_Wdst,
                        def1_bdst, def1_attn, def1_bias, def1_resW)
    h_def2 = _gat_layer(h_def1, src, dst, def2_Wsrc, def2_bsrc, def2_Wdst,
                        def2_bdst, def2_attn, def2_bias, None)
    z = jnp.concatenate([h_att1, h_def2, x], axis=1)
    h1 = _mlp1(z, W1, b1)
    h2 = _mlp2(h1, W2, b2)
    return _mlp34(h2, W3, b3, W4, b4)


# speedup vs baseline: 61.7460x; 58.3170x over previous
import functools

import jax
import jax.numpy as jnp
from jax import lax
from jax.experimental import pallas as pl
from jax.experimental.pallas import tpu as pltpu
from jax.experimental.pallas import tpu_sc as plsc

N = 10000
E = 640000
D = 16
H = 2
IN_F = 5 * D
HID = IN_F ** 2

NC, NS = 2, 16
NW = NC * NS
EB = 80
EPW = E // NW
NBLK = EPW // EB

BM = 1000


def _leaky(v, slope):
    return jnp.where(v > 0, v, slope * v)


def _elu(v):
    return jnp.where(v > 0, v, jnp.exp(v) - 1.0)


def _sc_edge_body(nh, fsrc_hbm, fdst_hbm, attn_hbm, srcb_hbm, dstb_hbm,
                  out_hbm, acc, attn_v, idxs_v, idxd_v, zb,
                  fs0, fd0, wb0, eb0, fs1, fd1, wb1, eb1,
                  sg0, ss0, sg1, ss1):
    fw = nh * 16
    aw = fw + 16
    c = lax.axis_index("c")
    s = lax.axis_index("s")
    w = s * NC + c
    zeros16 = jnp.zeros((16,), jnp.float32)

    for r in range(16):
        for j in range(aw // 16):
            zb[r, pl.ds(j * 16, 16)] = zeros16
    nz = jnp.where(s == NS - 1, 40, 39)

    def _zero_chunk(k, carry):
        off = pl.multiple_of(s * 624 + k * 16, 8)
        pltpu.sync_copy(zb, acc.at[pl.ds(off, 16)])
        return carry
    lax.fori_loop(0, nz, _zero_chunk, 0)

    def _wb_row(r, carry):
        wb0[r, pl.ds(fw, 16)] = zeros16
        wb1[r, pl.ds(fw, 16)] = zeros16
        return carry
    lax.fori_loop(0, EB, _wb_row, 0)

    pltpu.sync_copy(attn_hbm, attn_v)
    pltpu.sync_copy(srcb_hbm.at[w], idxs_v)
    pltpu.sync_copy(dstb_hbm.at[w], idxd_v)
    av = [attn_v[h, :] for h in range(nh)]

    plsc.subcore_barrier()

    slots = ((fs0, fd0, wb0, eb0, sg0, ss0), (fs1, fd1, wb1, eb1, sg1, ss1))

    def _fire_gathers(q, b):
        fs, fd, _, _, sg, _ = slots[b]
        pltpu.async_copy(fsrc_hbm.at[idxs_v.at[q]], fs, sg)
        pltpu.async_copy(fdst_hbm.at[idxd_v.at[q]], fd, sg)

    def _wait_gathers(q, b):
        fs, fd, _, _, sg, _ = slots[b]
        pltpu.make_async_copy(fsrc_hbm.at[idxs_v.at[q]], fs, sg).wait()
        pltpu.make_async_copy(fdst_hbm.at[idxd_v.at[q]], fd, sg).wait()

    iota16 = jnp.arange(16, dtype=jnp.int32)
    lane15 = iota16 == 15

    def _compute(q, b):
        fs, fd, wb, eb, _, _ = slots[b]

        def _edge_e(i, carry):
            for h in range(nh):
                a = fs[i, pl.ds(16 * h, 16)]
                t = a + fd[i, pl.ds(16 * h, 16)]
                t = jnp.maximum(t, 0.2 * t)
                cs = plsc.cumsum(t * av[h])
                plsc.store_scatter(
                    eb,
                    [jnp.full((16,), h, jnp.int32),
                     jnp.full((16,), i, jnp.int32)],
                    cs, mask=lane15)
            return carry
        lax.fori_loop(0, EB, _edge_e, 0)

        for h in range(nh):
            for j in range(EB // 16):
                sl = pl.ds(j * 16, 16)
                ex = jnp.exp(eb[h, sl])
                eb[h, sl] = ex
                plsc.store_scatter(
                    wb,
                    [j * 16 + iota16, jnp.full((16,), fw + h, jnp.int32)],
                    ex)

        def _group_w(g, carry):
            base = g * 16
            for h in range(nh):
                ex16 = eb[h, pl.ds(base, 16)]
                for k in range(16):
                    i = base + k
                    wb[i, pl.ds(16 * h, 16)] = (
                        ex16[k] * fs[i, pl.ds(16 * h, 16)])
            return carry
        lax.fori_loop(0, EB // 16, _group_w, 0)

    def _fire_scatter(q, b):
        _, _, wb, _, _, ss = slots[b]
        pltpu.async_copy(wb, acc.at[idxd_v.at[q]], ss, add=True)

    def _wait_scatter(q, b):
        _, _, wb, _, _, ss = slots[b]
        pltpu.make_async_copy(wb, acc.at[idxd_v.at[q]], ss).wait()

    _fire_gathers(0, 0)
    _fire_gathers(1, 1)

    def _outer(g, carry):
        for b in range(2):
            q = 2 * g + b
            _wait_gathers(q, b)

            @pl.when(q >= 2)
            def _():
                _wait_scatter(q - 2, b)

            _compute(q, b)
            _fire_scatter(q, b)

            @pl.when(q + 2 < NBLK)
            def _():
                _fire_gathers(q + 2, b)
        return carry
    lax.fori_loop(0, NBLK // 2, _outer, 0)

    _wait_scatter(NBLK - 2, 0)
    _wait_scatter(NBLK - 1, 1)
    plsc.subcore_barrier()

    def _dump_chunk(k, carry):
        off = pl.multiple_of(s * 624 + k * 16, 8)
        pltpu.sync_copy(acc.at[pl.ds(off, 16)], out_hbm.at[c, pl.ds(off, 16)])
        return carry
    lax.fori_loop(0, nz, _dump_chunk, 0)


def _sc_edge_pass(nh, fsrc, fdst, attn_cat, srcb, dstb):
    fw = nh * 16
    aw = fw + 16
    mesh = plsc.VectorSubcoreMesh(core_axis_name="c", subcore_axis_name="s")
    kfn = pl.kernel(
        functools.partial(_sc_edge_body, nh),
        out_type=jax.ShapeDtypeStruct((NC, N, aw), jnp.float32),
        mesh=mesh,
        compiler_params=pltpu.CompilerParams(
            needs_layout_passes=False, use_tc_tiling_on_sc=False),
        scratch_types=[
            pltpu.VMEM_SHARED((N, aw), jnp.float32),
            pltpu.VMEM((nh, 16), jnp.float32),
            pltpu.VMEM((NBLK, EB), jnp.int32),
            pltpu.VMEM((NBLK, EB), jnp.int32),
            pltpu.VMEM((16, aw), jnp.float32),
            pltpu.VMEM((EB, fw), jnp.float32),
            pltpu.VMEM((EB, fw), jnp.float32),
            pltpu.VMEM((EB, aw), jnp.float32),
            pltpu.VMEM((nh, EB), jnp.float32),
            pltpu.VMEM((EB, fw), jnp.float32),
            pltpu.VMEM((EB, fw), jnp.float32),
            pltpu.VMEM((EB, aw), jnp.float32),
            pltpu.VMEM((nh, EB), jnp.float32),
            pltpu.SemaphoreType.DMA,
            pltpu.SemaphoreType.DMA,
            pltpu.SemaphoreType.DMA,
            pltpu.SemaphoreType.DMA,
        ],
    )
    return kfn(fsrc, fdst, attn_cat, srcb, dstb)


def _prep1_body(x_ref, w_ref, b_ref, os_ref, od_ref):
    f = jnp.dot(x_ref[...], w_ref[...], preferred_element_type=jnp.float32)
    f = f + b_ref[...]
    os_ref[...] = f[:, :64]
    od_ref[...] = f[:, 64:]


def _prep1(x, Wcat, bcat):
    return pl.pallas_call(
        _prep1_body,
        grid=(N // BM,),
        in_specs=[
            pl.BlockSpec((BM, D), lambda m: (m, 0)),
            pl.BlockSpec((D, 128), lambda m: (0, 0)),
            pl.BlockSpec((1, 128), lambda m: (0, 0)),
        ],
        out_specs=[
            pl.BlockSpec((BM, 64), lambda m: (m, 0)),
            pl.BlockSpec((BM, 64), lambda m: (m, 0)),
        ],
        out_shape=[
            jax.ShapeDtypeStruct((N, 64), jnp.float32),
            jax.ShapeDtypeStruct((N, 64), jnp.float32),
        ],
    )(x, Wcat, bcat)


def _finish1_body(acc_ref, x_ref, rw_ref, bc_ref, w2c_ref, b2c_ref,
                  ha_ref, hd_ref, f2s_ref, f2d_ref):
    a = acc_ref[0] + acc_ref[1]
    res = jnp.dot(x_ref[...], rw_ref[...], preferred_element_type=jnp.float32)
    parts = []
    for h in range(4):
        den = jnp.maximum(a[:, 64 + h:65 + h], 1e-16)
        parts.append(a[:, 16 * h:16 * h + 16] / den)
    rst = jnp.concatenate(parts, axis=1) + res + bc_ref[...]
    hcat = _elu(rst)
    ha_ref[...] = hcat[:, :32]
    hd = hcat[:, 32:]
    hd_ref[...] = hd
    f2 = jnp.dot(hd, w2c_ref[...], preferred_element_type=jnp.float32)
    f2 = f2 + b2c_ref[...]
    f2s_ref[...] = f2[:, :32]
    f2d_ref[...] = f2[:, 32:]


def _finish1(acc1, x, rWcat, biascat, W2cat, b2cat):
    return pl.pallas_call(
        _finish1_body,
        grid=(N // BM,),
        in_specs=[
            pl.BlockSpec((NC, BM, 80), lambda m: (0, m, 0)),
            pl.BlockSpec((BM, D), lambda m: (m, 0)),
            pl.BlockSpec((D, 64), lambda m: (0, 0)),
            pl.BlockSpec((1, 64), lambda m: (0, 0)),
            pl.BlockSpec((32, 64), lambda m: (0, 0)),
            pl.BlockSpec((1, 64), lambda m: (0, 0)),
        ],
        out_specs=[
            pl.BlockSpec((BM, 32), lambda m: (m, 0)),
            pl.BlockSpec((BM, 32), lambda m: (m, 0)),
            pl.BlockSpec((BM, 32), lambda m: (m, 0)),
            pl.BlockSpec((BM, 32), lambda m: (m, 0)),
        ],
        out_shape=[jax.ShapeDtypeStruct((N, 32), jnp.float32)] * 4,
    )(acc1, x, rWcat, biascat, W2cat, b2cat)


def _finish2_body(acc_ref, ha_ref, hd_ref, x_ref, b2_ref, w1_ref, b1_ref,
                  o_ref):
    a = acc_ref[0] + acc_ref[1]
    parts = []
    for h in range(2):
        den = jnp.maximum(a[:, 32 + h:33 + h], 1e-16)
        parts.append(a[:, 16 * h:16 * h + 16] / den)
    rst = jnp.concatenate(parts, axis=1) + hd_ref[...] + b2_ref[...]
    hd2 = _elu(rst)
    z = jnp.concatenate([ha_ref[...], hd2, x_ref[...]], axis=1)
    acc = jnp.dot(z, w1_ref[...], preferred_element_type=jnp.float32)
    o_ref[...] = _leaky(acc + b1_ref[...], 0.01)


def _finish2_mlp1(acc2, hA, hD, x, bias2, W1, b1):
    return pl.pallas_call(
        _finish2_body,
        grid=(N // BM,),
        in_specs=[
            pl.BlockSpec((NC, BM, 48), lambda m: (0, m, 0)),
            pl.BlockSpec((BM, 32), lambda m: (m, 0)),
            pl.BlockSpec((BM, 32), lambda m: (m, 0)),
            pl.BlockSpec((BM, D), lambda m: (m, 0)),
            pl.BlockSpec((1, 32), lambda m: (0, 0)),
            pl.BlockSpec((IN_F, HID), lambda m: (0, 0)),
            pl.BlockSpec((1, HID), lambda m: (0, 0)),
        ],
        out_specs=pl.BlockSpec((BM, HID), lambda m: (m, 0)),
        out_shape=jax.ShapeDtypeStruct((N, HID), jnp.float32),
    )(acc2, hA, hD, x, bias2, W1, b1.reshape(1, HID))


def _mlp2_body(h_ref, w_ref, b_ref, o_ref, acc_ref, *, nk):
    k = pl.program_id(2)

    @pl.when(k == 0)
    def _():
        acc_ref[...] = jnp.zeros_like(acc_ref)

    acc_ref[...] += jnp.dot(h_ref[...], w_ref[...],
                            preferred_element_type=jnp.float32)

    @pl.when(k == nk - 1)
    def _():
        o_ref[...] = _leaky(acc_ref[...] + b_ref[...], 0.01)


def _mlp2(h1, W2, b2):
    bn, bk = 1280, 640
    nk = HID // bk
    return pl.pallas_call(
        functools.partial(_mlp2_body, nk=nk),
        grid=(N // BM, HID // bn, nk),
        in_specs=[
            pl.BlockSpec((BM, bk), lambda m, n, k: (m, k)),
            pl.BlockSpec((bk, bn), lambda m, n, k: (k, n)),
            pl.BlockSpec((1, bn), lambda m, n, k: (0, n)),
        ],
        out_specs=pl.BlockSpec((BM, bn), lambda m, n, k: (m, n)),
        out_shape=jax.ShapeDtypeStruct((N, HID), jnp.float32),
        scratch_shapes=[pltpu.VMEM((BM, bn), jnp.float32)],
    )(h1, W2, b2.reshape(1, HID))


def _mlp34_body(h_ref, w3_ref, b3_ref, w4_ref, b4_ref, o_ref):
    h3 = jnp.dot(h_ref[...], w3_ref[...], preferred_element_type=jnp.float32)
    h3 = _leaky(h3 + b3_ref[...], 0.01)
    h4 = jnp.dot(h3, w4_ref[...], preferred_element_type=jnp.float32)
    o_ref[...] = jax.nn.sigmoid(h4 + b4_ref[...])


def _mlp34(h2, W3, b3, W4, b4):
    return pl.pallas_call(
        _mlp34_body,
        grid=(N // BM,),
        in_specs=[
            pl.BlockSpec((BM, HID), lambda m: (m, 0)),
            pl.BlockSpec((HID, IN_F), lambda m: (0, 0)),
            pl.BlockSpec((1, IN_F), lambda m: (0, 0)),
            pl.BlockSpec((IN_F, 1), lambda m: (0, 0)),
            pl.BlockSpec((1, 1), lambda m: (0, 0)),
        ],
        out_specs=pl.BlockSpec((BM, 1), lambda m: (m, 0)),
        out_shape=jax.ShapeDtypeStruct((N, 1), jnp.float32),
    )(h2, W3, b3.reshape(1, IN_F), W4, b4.reshape(1, 1))


def kernel(x, edge_index,
           att1_Wsrc, att1_bsrc, att1_Wdst, att1_bdst, att1_attn, att1_bias, att1_resW,
           def1_Wsrc, def1_bsrc, def1_Wdst, def1_bdst, def1_attn, def1_bias, def1_resW,
           def2_Wsrc, def2_bsrc, def2_Wdst, def2_bdst, def2_attn, def2_bias,
           W1, b1, W2, b2, W3, b3, W4, b4):
    srcb = edge_index[0].reshape(NW, NBLK, EB)
    dstb = edge_index[1].reshape(NW, NBLK, EB)

    Wcat = jnp.concatenate(
        [att1_Wsrc, def1_Wsrc, att1_Wdst, def1_Wdst], axis=1)
    bcat = jnp.concatenate(
        [att1_bsrc, def1_bsrc, att1_bdst, def1_bdst]).reshape(1, 128)
    fsrc1, fdst1 = _prep1(x, Wcat, bcat)
    attn1 = jnp.concatenate([att1_attn, def1_attn], axis=0)
    acc1 = _sc_edge_pass(4, fsrc1, fdst1, attn1, srcb, dstb)

    rWcat = jnp.concatenate([att1_resW, def1_resW], axis=1)
    biascat = jnp.concatenate([att1_bias, def1_bias]).reshape(1, 64)
    W2cat = jnp.concatenate([def2_Wsrc, def2_Wdst], axis=1)
    b2cat = jnp.concatenate([def2_bsrc, def2_bdst]).reshape(1, 64)
    hA, hD, f2s, f2d = _finish1(acc1, x, rWcat, biascat, W2cat, b2cat)

    acc2 = _sc_edge_pass(2, f2s, f2d, def2_attn, srcb, dstb)

    h1 = _finish2_mlp1(acc2, hA, hD, x, def2_bias.reshape(1, 32), W1, b1)
    h2 = _mlp2(h1, W2, b2)
    return _mlp34(h2, W3, b3, W4, b4)


# speedup vs baseline: 62.3951x; 1.0105x over previous
import functools

import jax
import jax.numpy as jnp
from jax import lax
from jax.experimental import pallas as pl
from jax.experimental.pallas import tpu as pltpu
from jax.experimental.pallas import tpu_sc as plsc

N = 10000
E = 640000
D = 16
H = 2
IN_F = 5 * D
HID = IN_F ** 2

NC, NS = 2, 16
NW = NC * NS
EB = 80
EPW = E // NW
NBLK = EPW // EB

BM = 1000


def _leaky(v, slope):
    return jnp.where(v > 0, v, slope * v)


def _elu(v):
    return jnp.where(v > 0, v, jnp.exp(v) - 1.0)


def _sc_edge_body(nh, fsrc_hbm, fdst_hbm, attn_hbm, srcb_hbm, dstb_hbm,
                  out_hbm, acc, attn_v, idxs_v, idxd_v, zb,
                  fs0, fd0, wb0, eb0, fs1, fd1, wb1, eb1,
                  sg0, ss0, sg1, ss1):
    fw = nh * 16
    aw = fw + 16
    c = lax.axis_index("c")
    s = lax.axis_index("s")
    w = s * NC + c
    zeros16 = jnp.zeros((16,), jnp.float32)

    for r in range(16):
        for j in range(aw // 16):
            zb[r, pl.ds(j * 16, 16)] = zeros16
    nz = jnp.where(s == NS - 1, 40, 39)

    def _zero_chunk(k, carry):
        off = pl.multiple_of(s * 624 + k * 16, 8)
        pltpu.sync_copy(zb, acc.at[pl.ds(off, 16)])
        return carry
    lax.fori_loop(0, nz, _zero_chunk, 0)

    def _wb_row(r, carry):
        wb0[r, pl.ds(fw, 16)] = zeros16
        wb1[r, pl.ds(fw, 16)] = zeros16
        return carry
    lax.fori_loop(0, EB, _wb_row, 0)

    pltpu.sync_copy(attn_hbm, attn_v)
    pltpu.sync_copy(srcb_hbm.at[w], idxs_v)
    pltpu.sync_copy(dstb_hbm.at[w], idxd_v)
    av = [attn_v[h, :] for h in range(nh)]

    plsc.subcore_barrier()

    slots = ((fs0, fd0, wb0, eb0, sg0, ss0), (fs1, fd1, wb1, eb1, sg1, ss1))

    def _fire_gathers(q, b):
        fs, fd, _, _, sg, _ = slots[b]
        pltpu.async_copy(fsrc_hbm.at[idxs_v.at[q]], fs, sg)
        pltpu.async_copy(fdst_hbm.at[idxd_v.at[q]], fd, sg)

    def _wait_gathers(q, b):
        fs, fd, _, _, sg, _ = slots[b]
        pltpu.make_async_copy(fsrc_hbm.at[idxs_v.at[q]], fs, sg).wait()
        pltpu.make_async_copy(fdst_hbm.at[idxd_v.at[q]], fd, sg).wait()

    iota16 = jnp.arange(16, dtype=jnp.int32)
    lane15 = iota16 == 15

    def _compute(q, b):
        fs, fd, wb, eb, _, _ = slots[b]

        def _edge_e(i, carry):
            for h in range(nh):
                a = fs[i, pl.ds(16 * h, 16)]
                t = a + fd[i, pl.ds(16 * h, 16)]
                t = jnp.maximum(t, 0.2 * t)
                cs = plsc.cumsum(t * av[h])
                plsc.store_scatter(
                    eb,
                    [jnp.full((16,), h, jnp.int32),
                     jnp.full((16,), i, jnp.int32)],
                    cs, mask=lane15)
            return carry
        lax.fori_loop(0, EB, _edge_e, 0, unroll=4)

        for h in range(nh):
            for j in range(EB // 16):
                sl = pl.ds(j * 16, 16)
                ex = jnp.exp(eb[h, sl])
                eb[h, sl] = ex
                plsc.store_scatter(
                    wb,
                    [j * 16 + iota16, jnp.full((16,), fw + h, jnp.int32)],
                    ex)

        def _group_w(g, carry):
            base = g * 16
            for h in range(nh):
                ex16 = eb[h, pl.ds(base, 16)]
                for k in range(16):
                    i = base + k
                    wb[i, pl.ds(16 * h, 16)] = (
                        ex16[k] * fs[i, pl.ds(16 * h, 16)])
            return carry
        lax.fori_loop(0, EB // 16, _group_w, 0)

    def _fire_scatter(q, b):
        _, _, wb, _, _, ss = slots[b]
        pltpu.async_copy(wb, acc.at[idxd_v.at[q]], ss, add=True)

    def _wait_scatter(q, b):
        _, _, wb, _, _, ss = slots[b]
        pltpu.make_async_copy(wb, acc.at[idxd_v.at[q]], ss).wait()

    _fire_gathers(0, 0)
    _fire_gathers(1, 1)

    def _outer(g, carry):
        for b in range(2):
            q = 2 * g + b
            _wait_gathers(q, b)

            @pl.when(q >= 2)
            def _():
                _wait_scatter(q - 2, b)

            _compute(q, b)
            _fire_scatter(q, b)

            @pl.when(q + 2 < NBLK)
            def _():
                _fire_gathers(q + 2, b)
        return carry
    lax.fori_loop(0, NBLK // 2, _outer, 0)

    _wait_scatter(NBLK - 2, 0)
    _wait_scatter(NBLK - 1, 1)
    plsc.subcore_barrier()

    def _dump_chunk(k, carry):
        off = pl.multiple_of(s * 624 + k * 16, 8)
        pltpu.sync_copy(acc.at[pl.ds(off, 16)], out_hbm.at[c, pl.ds(off, 16)])
        return carry
    lax.fori_loop(0, nz, _dump_chunk, 0)


def _sc_edge_pass(nh, fsrc, fdst, attn_cat, srcb, dstb):
    fw = nh * 16
    aw = fw + 16
    mesh = plsc.VectorSubcoreMesh(core_axis_name="c", subcore_axis_name="s")
    kfn = pl.kernel(
        functools.partial(_sc_edge_body, nh),
        out_type=jax.ShapeDtypeStruct((NC, N, aw), jnp.float32),
        mesh=mesh,
        compiler_params=pltpu.CompilerParams(
            needs_layout_passes=False, use_tc_tiling_on_sc=False),
        scratch_types=[
            pltpu.VMEM_SHARED((N, aw), jnp.float32),
            pltpu.VMEM((nh, 16), jnp.float32),
            pltpu.VMEM((NBLK, EB), jnp.int32),
            pltpu.VMEM((NBLK, EB), jnp.int32),
            pltpu.VMEM((16, aw), jnp.float32),
            pltpu.VMEM((EB, fw), jnp.float32),
            pltpu.VMEM((EB, fw), jnp.float32),
            pltpu.VMEM((EB, aw), jnp.float32),
            pltpu.VMEM((nh, EB), jnp.float32),
            pltpu.VMEM((EB, fw), jnp.float32),
            pltpu.VMEM((EB, fw), jnp.float32),
            pltpu.VMEM((EB, aw), jnp.float32),
            pltpu.VMEM((nh, EB), jnp.float32),
            pltpu.SemaphoreType.DMA,
            pltpu.SemaphoreType.DMA,
            pltpu.SemaphoreType.DMA,
            pltpu.SemaphoreType.DMA,
        ],
    )
    return kfn(fsrc, fdst, attn_cat, srcb, dstb)


def _prep1_body(x_ref, w_ref, b_ref, os_ref, od_ref):
    f = jnp.dot(x_ref[...], w_ref[...], preferred_element_type=jnp.float32)
    f = f + b_ref[...]
    os_ref[...] = f[:, :64]
    od_ref[...] = f[:, 64:]


def _prep1(x, Wcat, bcat):
    return pl.pallas_call(
        _prep1_body,
        grid=(N // BM,),
        in_specs=[
            pl.BlockSpec((BM, D), lambda m: (m, 0)),
            pl.BlockSpec((D, 128), lambda m: (0, 0)),
            pl.BlockSpec((1, 128), lambda m: (0, 0)),
        ],
        out_specs=[
            pl.BlockSpec((BM, 64), lambda m: (m, 0)),
            pl.BlockSpec((BM, 64), lambda m: (m, 0)),
        ],
        out_shape=[
            jax.ShapeDtypeStruct((N, 64), jnp.float32),
            jax.ShapeDtypeStruct((N, 64), jnp.float32),
        ],
    )(x, Wcat, bcat)


def _finish1_body(acc_ref, x_ref, rw_ref, bc_ref, w2c_ref, b2c_ref,
                  ha_ref, hd_ref, f2s_ref, f2d_ref):
    a = acc_ref[0] + acc_ref[1]
    res = jnp.dot(x_ref[...], rw_ref[...], preferred_element_type=jnp.float32)
    parts = []
    for h in range(4):
        den = jnp.maximum(a[:, 64 + h:65 + h], 1e-16)
        parts.append(a[:, 16 * h:16 * h + 16] / den)
    rst = jnp.concatenate(parts, axis=1) + res + bc_ref[...]
    hcat = _elu(rst)
    ha_ref[...] = hcat[:, :32]
    hd = hcat[:, 32:]
    hd_ref[...] = hd
    f2 = jnp.dot(hd, w2c_ref[...], preferred_element_type=jnp.float32)
    f2 = f2 + b2c_ref[...]
    f2s_ref[...] = f2[:, :32]
    f2d_ref[...] = f2[:, 32:]


def _finish1(acc1, x, rWcat, biascat, W2cat, b2cat):
    return pl.pallas_call(
        _finish1_body,
        grid=(N // BM,),
        in_specs=[
            pl.BlockSpec((NC, BM, 80), lambda m: (0, m, 0)),
            pl.BlockSpec((BM, D), lambda m: (m, 0)),
            pl.BlockSpec((D, 64), lambda m: (0, 0)),
            pl.BlockSpec((1, 64), lambda m: (0, 0)),
            pl.BlockSpec((32, 64), lambda m: (0, 0)),
            pl.BlockSpec((1, 64), lambda m: (0, 0)),
        ],
        out_specs=[
            pl.BlockSpec((BM, 32), lambda m: (m, 0)),
            pl.BlockSpec((BM, 32), lambda m: (m, 0)),
            pl.BlockSpec((BM, 32), lambda m: (m, 0)),
            pl.BlockSpec((BM, 32), lambda m: (m, 0)),
        ],
        out_shape=[jax.ShapeDtypeStruct((N, 32), jnp.float32)] * 4,
    )(acc1, x, rWcat, biascat, W2cat, b2cat)


def _finish2_body(acc_ref, ha_ref, hd_ref, x_ref, b2_ref, w1_ref, b1_ref,
                  o_ref):
    a = acc_ref[0] + acc_ref[1]
    parts = []
    for h in range(2):
        den = jnp.maximum(a[:, 32 + h:33 + h], 1e-16)
        parts.append(a[:, 16 * h:16 * h + 16] / den)
    rst = jnp.concatenate(parts, axis=1) + hd_ref[...] + b2_ref[...]
    hd2 = _elu(rst)
    z = jnp.concatenate([ha_ref[...], hd2, x_ref[...]], axis=1)
    acc = jnp.dot(z, w1_ref[...], preferred_element_type=jnp.float32)
    o_ref[...] = _leaky(acc + b1_ref[...], 0.01)


def _finish2_mlp1(acc2, hA, hD, x, bias2, W1, b1):
    return pl.pallas_call(
        _finish2_body,
        grid=(N // BM,),
        in_specs=[
            pl.BlockSpec((NC, BM, 48), lambda m: (0, m, 0)),
            pl.BlockSpec((BM, 32), lambda m: (m, 0)),
            pl.BlockSpec((BM, 32), lambda m: (m, 0)),
            pl.BlockSpec((BM, D), lambda m: (m, 0)),
            pl.BlockSpec((1, 32), lambda m: (0, 0)),
            pl.BlockSpec((IN_F, HID), lambda m: (0, 0)),
            pl.BlockSpec((1, HID), lambda m: (0, 0)),
        ],
        out_specs=pl.BlockSpec((BM, HID), lambda m: (m, 0)),
        out_shape=jax.ShapeDtypeStruct((N, HID), jnp.float32),
    )(acc2, hA, hD, x, bias2, W1, b1.reshape(1, HID))


def _mlp2_body(h_ref, w_ref, b_ref, o_ref, acc_ref, *, nk):
    k = pl.program_id(2)

    @pl.when(k == 0)
    def _():
        acc_ref[...] = jnp.zeros_like(acc_ref)

    acc_ref[...] += jnp.dot(h_ref[...], w_ref[...],
                            preferred_element_type=jnp.float32)

    @pl.when(k == nk - 1)
    def _():
        o_ref[...] = _leaky(acc_ref[...] + b_ref[...], 0.01)


def _mlp2(h1, W2, b2):
    bn, bk = 1280, 640
    nk = HID // bk
    return pl.pallas_call(
        functools.partial(_mlp2_body, nk=nk),
        grid=(N // BM, HID // bn, nk),
        in_specs=[
            pl.BlockSpec((BM, bk), lambda m, n, k: (m, k)),
            pl.BlockSpec((bk, bn), lambda m, n, k: (k, n)),
            pl.BlockSpec((1, bn), lambda m, n, k: (0, n)),
        ],
        out_specs=pl.BlockSpec((BM, bn), lambda m, n, k: (m, n)),
        out_shape=jax.ShapeDtypeStruct((N, HID), jnp.float32),
        scratch_shapes=[pltpu.VMEM((BM, bn), jnp.float32)],
    )(h1, W2, b2.reshape(1, HID))


def _mlp34_body(h_ref, w3_ref, b3_ref, w4_ref, b4_ref, o_ref):
    h3 = jnp.dot(h_ref[...], w3_ref[...], preferred_element_type=jnp.float32)
    h3 = _leaky(h3 + b3_ref[...], 0.01)
    h4 = jnp.dot(h3, w4_ref[...], preferred_element_type=jnp.float32)
    o_ref[...] = jax.nn.sigmoid(h4 + b4_ref[...])


def _mlp34(h2, W3, b3, W4, b4):
    return pl.pallas_call(
        _mlp34_body,
        grid=(N // BM,),
        in_specs=[
            pl.BlockSpec((BM, HID), lambda m: (m, 0)),
            pl.BlockSpec((HID, IN_F), lambda m: (0, 0)),
            pl.BlockSpec((1, IN_F), lambda m: (0, 0)),
            pl.BlockSpec((IN_F, 1), lambda m: (0, 0)),
            pl.BlockSpec((1, 1), lambda m: (0, 0)),
        ],
        out_specs=pl.BlockSpec((BM, 1), lambda m: (m, 0)),
        out_shape=jax.ShapeDtypeStruct((N, 1), jnp.float32),
    )(h2, W3, b3.reshape(1, IN_F), W4, b4.reshape(1, 1))


def kernel(x, edge_index,
           att1_Wsrc, att1_bsrc, att1_Wdst, att1_bdst, att1_attn, att1_bias, att1_resW,
           def1_Wsrc, def1_bsrc, def1_Wdst, def1_bdst, def1_attn, def1_bias, def1_resW,
           def2_Wsrc, def2_bsrc, def2_Wdst, def2_bdst, def2_attn, def2_bias,
           W1, b1, W2, b2, W3, b3, W4, b4):
    srcb = edge_index[0].reshape(NW, NBLK, EB)
    dstb = edge_index[1].reshape(NW, NBLK, EB)

    Wcat = jnp.concatenate(
        [att1_Wsrc, def1_Wsrc, att1_Wdst, def1_Wdst], axis=1)
    bcat = jnp.concatenate(
        [att1_bsrc, def1_bsrc, att1_bdst, def1_bdst]).reshape(1, 128)
    fsrc1, fdst1 = _prep1(x, Wcat, bcat)
    attn1 = jnp.concatenate([att1_attn, def1_attn], axis=0)
    acc1 = _sc_edge_pass(4, fsrc1, fdst1, attn1, srcb, dstb)

    rWcat = jnp.concatenate([att1_resW, def1_resW], axis=1)
    biascat = jnp.concatenate([att1_bias, def1_bias]).reshape(1, 64)
    W2cat = jnp.concatenate([def2_Wsrc, def2_Wdst], axis=1)
    b2cat = jnp.concatenate([def2_bsrc, def2_bdst]).reshape(1, 64)
    hA, hD, f2s, f2d = _finish1(acc1, x, rWcat, biascat, W2cat, b2cat)

    acc2 = _sc_edge_pass(2, f2s, f2d, def2_attn, srcb, dstb)

    h1 = _finish2_mlp1(acc2, hA, hD, x, def2_bias.reshape(1, 32), W1, b1)
    h2 = _mlp2(h1, W2, b2)
    return _mlp34(h2, W3, b3, W4, b4)


# speedup vs baseline: 101.5920x; 1.6282x over previous
import functools

import jax
import jax.numpy as jnp
from jax import lax
from jax.experimental import pallas as pl
from jax.experimental.pallas import tpu as pltpu
from jax.experimental.pallas import tpu_sc as plsc

N = 10000
E = 640000
D = 16
H = 2
IN_F = 5 * D
HID = IN_F ** 2

NC, NS = 2, 16
NW = NC * NS
EB = 80
EPW = E // NW
NBLK = EPW // EB

BM = 1000


def _leaky(v, slope):
    return jnp.where(v > 0, v, slope * v)


def _elu(v):
    return jnp.where(v > 0, v, jnp.exp(v) - 1.0)


def _sc_edge_body(nh, fsrc_hbm, fdst_hbm, attn_hbm, srcb_hbm, dstb_hbm,
                  out_hbm, acc, attn_v, idxs_v, idxd_v, zb,
                  fs0, fd0, wb0, eb0, fs1, fd1, wb1, eb1,
                  sg0, ss0, sg1, ss1):
    fw = nh * 16
    aw = fw + 16
    c = lax.axis_index("c")
    s = lax.axis_index("s")
    w = s * NC + c
    zeros16 = jnp.zeros((16,), jnp.float32)

    for r in range(16):
        for j in range(aw // 16):
            zb[r, pl.ds(j * 16, 16)] = zeros16
    nz = jnp.where(s == NS - 1, 40, 39)

    def _zero_chunk(k, carry):
        off = pl.multiple_of(s * 624 + k * 16, 8)
        pltpu.sync_copy(zb, acc.at[pl.ds(off, 16)])
        return carry
    lax.fori_loop(0, nz, _zero_chunk, 0)

    def _wb_row(r, carry):
        wb0[r, pl.ds(fw, 16)] = zeros16
        wb1[r, pl.ds(fw, 16)] = zeros16
        return carry
    lax.fori_loop(0, EB, _wb_row, 0)

    pltpu.sync_copy(attn_hbm, attn_v)
    pltpu.sync_copy(srcb_hbm.at[w], idxs_v)
    pltpu.sync_copy(dstb_hbm.at[w], idxd_v)
    av = [attn_v[h, :] for h in range(nh)]

    plsc.subcore_barrier()

    slots = ((fs0, fd0, wb0, eb0, sg0, ss0), (fs1, fd1, wb1, eb1, sg1, ss1))

    def _fire_gathers(q, b):
        fs, fd, _, _, sg, _ = slots[b]
        pltpu.async_copy(fsrc_hbm.at[idxs_v.at[q]], fs, sg)
        pltpu.async_copy(fdst_hbm.at[idxd_v.at[q]], fd, sg)

    def _wait_gathers(q, b):
        fs, fd, _, _, sg, _ = slots[b]
        pltpu.make_async_copy(fsrc_hbm.at[idxs_v.at[q]], fs, sg).wait()
        pltpu.make_async_copy(fdst_hbm.at[idxd_v.at[q]], fd, sg).wait()

    iota16 = jnp.arange(16, dtype=jnp.int32)
    lane15 = iota16 == 15

    def _compute(q, b):
        fs, fd, wb, eb, _, _ = slots[b]

        @plsc.parallel_loop(0, EB, 1, unroll=4)
        def _edge_e(i):
            for h in range(nh):
                a = fs[i, pl.ds(16 * h, 16)]
                t = a + fd[i, pl.ds(16 * h, 16)]
                t = jnp.maximum(t, 0.2 * t)
                cs = plsc.cumsum(t * av[h])
                plsc.store_scatter(
                    eb,
                    [jnp.full((16,), h, jnp.int32),
                     jnp.full((16,), i, jnp.int32)],
                    cs, mask=lane15)

        for h in range(nh):
            for j in range(EB // 16):
                sl = pl.ds(j * 16, 16)
                ex = jnp.exp(eb[h, sl])
                eb[h, sl] = ex
                plsc.store_scatter(
                    wb,
                    [j * 16 + iota16, jnp.full((16,), fw + h, jnp.int32)],
                    ex)

        @plsc.parallel_loop(0, EB // 16, 1)
        def _group_w(g):
            base = g * 16
            for h in range(nh):
                ex16 = eb[h, pl.ds(base, 16)]
                for k in range(16):
                    i = base + k
                    wb[i, pl.ds(16 * h, 16)] = (
                        ex16[k] * fs[i, pl.ds(16 * h, 16)])

    def _fire_scatter(q, b):
        _, _, wb, _, _, ss = slots[b]
        pltpu.async_copy(wb, acc.at[idxd_v.at[q]], ss, add=True)

    def _wait_scatter(q, b):
        _, _, wb, _, _, ss = slots[b]
        pltpu.make_async_copy(wb, acc.at[idxd_v.at[q]], ss).wait()

    _fire_gathers(0, 0)
    _fire_gathers(1, 1)

    def _outer(g, carry):
        for b in range(2):
            q = 2 * g + b
            _wait_gathers(q, b)

            @pl.when(q >= 2)
            def _():
                _wait_scatter(q - 2, b)

            _compute(q, b)
            _fire_scatter(q, b)

            @pl.when(q + 2 < NBLK)
            def _():
                _fire_gathers(q + 2, b)
        return carry
    lax.fori_loop(0, NBLK // 2, _outer, 0)

    _wait_scatter(NBLK - 2, 0)
    _wait_scatter(NBLK - 1, 1)
    plsc.subcore_barrier()

    def _dump_chunk(k, carry):
        off = pl.multiple_of(s * 624 + k * 16, 8)
        pltpu.sync_copy(acc.at[pl.ds(off, 16)], out_hbm.at[c, pl.ds(off, 16)])
        return carry
    lax.fori_loop(0, nz, _dump_chunk, 0)


def _sc_edge_pass(nh, fsrc, fdst, attn_cat, srcb, dstb):
    fw = nh * 16
    aw = fw + 16
    mesh = plsc.VectorSubcoreMesh(core_axis_name="c", subcore_axis_name="s")
    kfn = pl.kernel(
        functools.partial(_sc_edge_body, nh),
        out_type=jax.ShapeDtypeStruct((NC, N, aw), jnp.float32),
        mesh=mesh,
        compiler_params=pltpu.CompilerParams(
            needs_layout_passes=False, use_tc_tiling_on_sc=False),
        scratch_types=[
            pltpu.VMEM_SHARED((N, aw), jnp.float32),
            pltpu.VMEM((nh, 16), jnp.float32),
            pltpu.VMEM((NBLK, EB), jnp.int32),
            pltpu.VMEM((NBLK, EB), jnp.int32),
            pltpu.VMEM((16, aw), jnp.float32),
            pltpu.VMEM((EB, fw), jnp.float32),
            pltpu.VMEM((EB, fw), jnp.float32),
            pltpu.VMEM((EB, aw), jnp.float32),
            pltpu.VMEM((nh, EB), jnp.float32),
            pltpu.VMEM((EB, fw), jnp.float32),
            pltpu.VMEM((EB, fw), jnp.float32),
            pltpu.VMEM((EB, aw), jnp.float32),
            pltpu.VMEM((nh, EB), jnp.float32),
            pltpu.SemaphoreType.DMA,
            pltpu.SemaphoreType.DMA,
            pltpu.SemaphoreType.DMA,
            pltpu.SemaphoreType.DMA,
        ],
    )
    return kfn(fsrc, fdst, attn_cat, srcb, dstb)


def _prep1_body(x_ref, w_ref, b_ref, os_ref, od_ref):
    f = jnp.dot(x_ref[...], w_ref[...], preferred_element_type=jnp.float32)
    f = f + b_ref[...]
    os_ref[...] = f[:, :64]
    od_ref[...] = f[:, 64:]


def _prep1(x, Wcat, bcat):
    return pl.pallas_call(
        _prep1_body,
        grid=(N // BM,),
        in_specs=[
            pl.BlockSpec((BM, D), lambda m: (m, 0)),
            pl.BlockSpec((D, 128), lambda m: (0, 0)),
            pl.BlockSpec((1, 128), lambda m: (0, 0)),
        ],
        out_specs=[
            pl.BlockSpec((BM, 64), lambda m: (m, 0)),
            pl.BlockSpec((BM, 64), lambda m: (m, 0)),
        ],
        out_shape=[
            jax.ShapeDtypeStruct((N, 64), jnp.float32),
            jax.ShapeDtypeStruct((N, 64), jnp.float32),
        ],
    )(x, Wcat, bcat)


def _finish1_body(acc_ref, x_ref, rw_ref, bc_ref, w2c_ref, b2c_ref,
                  ha_ref, hd_ref, f2s_ref, f2d_ref):
    a = acc_ref[0] + acc_ref[1]
    res = jnp.dot(x_ref[...], rw_ref[...], preferred_element_type=jnp.float32)
    parts = []
    for h in range(4):
        den = jnp.maximum(a[:, 64 + h:65 + h], 1e-16)
        parts.append(a[:, 16 * h:16 * h + 16] / den)
    rst = jnp.concatenate(parts, axis=1) + res + bc_ref[...]
    hcat = _elu(rst)
    ha_ref[...] = hcat[:, :32]
    hd = hcat[:, 32:]
    hd_ref[...] = hd
    f2 = jnp.dot(hd, w2c_ref[...], preferred_element_type=jnp.float32)
    f2 = f2 + b2c_ref[...]
    f2s_ref[...] = f2[:, :32]
    f2d_ref[...] = f2[:, 32:]


def _finish1(acc1, x, rWcat, biascat, W2cat, b2cat):
    return pl.pallas_call(
        _finish1_body,
        grid=(N // BM,),
        in_specs=[
            pl.BlockSpec((NC, BM, 80), lambda m: (0, m, 0)),
            pl.BlockSpec((BM, D), lambda m: (m, 0)),
            pl.BlockSpec((D, 64), lambda m: (0, 0)),
            pl.BlockSpec((1, 64), lambda m: (0, 0)),
            pl.BlockSpec((32, 64), lambda m: (0, 0)),
            pl.BlockSpec((1, 64), lambda m: (0, 0)),
        ],
        out_specs=[
            pl.BlockSpec((BM, 32), lambda m: (m, 0)),
            pl.BlockSpec((BM, 32), lambda m: (m, 0)),
            pl.BlockSpec((BM, 32), lambda m: (m, 0)),
            pl.BlockSpec((BM, 32), lambda m: (m, 0)),
        ],
        out_shape=[jax.ShapeDtypeStruct((N, 32), jnp.float32)] * 4,
    )(acc1, x, rWcat, biascat, W2cat, b2cat)


def _finish2_body(acc_ref, ha_ref, hd_ref, x_ref, b2_ref, w1_ref, b1_ref,
                  o_ref):
    a = acc_ref[0] + acc_ref[1]
    parts = []
    for h in range(2):
        den = jnp.maximum(a[:, 32 + h:33 + h], 1e-16)
        parts.append(a[:, 16 * h:16 * h + 16] / den)
    rst = jnp.concatenate(parts, axis=1) + hd_ref[...] + b2_ref[...]
    hd2 = _elu(rst)
    z = jnp.concatenate([ha_ref[...], hd2, x_ref[...]], axis=1)
    acc = jnp.dot(z, w1_ref[...], preferred_element_type=jnp.float32)
    o_ref[...] = _leaky(acc + b1_ref[...], 0.01)


def _finish2_mlp1(acc2, hA, hD, x, bias2, W1, b1):
    return pl.pallas_call(
        _finish2_body,
        grid=(N // BM,),
        in_specs=[
            pl.BlockSpec((NC, BM, 48), lambda m: (0, m, 0)),
            pl.BlockSpec((BM, 32), lambda m: (m, 0)),
            pl.BlockSpec((BM, 32), lambda m: (m, 0)),
            pl.BlockSpec((BM, D), lambda m: (m, 0)),
            pl.BlockSpec((1, 32), lambda m: (0, 0)),
            pl.BlockSpec((IN_F, HID), lambda m: (0, 0)),
            pl.BlockSpec((1, HID), lambda m: (0, 0)),
        ],
        out_specs=pl.BlockSpec((BM, HID), lambda m: (m, 0)),
        out_shape=jax.ShapeDtypeStruct((N, HID), jnp.float32),
    )(acc2, hA, hD, x, bias2, W1, b1.reshape(1, HID))


def _mlp2_body(h_ref, w_ref, b_ref, o_ref, acc_ref, *, nk):
    k = pl.program_id(2)

    @pl.when(k == 0)
    def _():
        acc_ref[...] = jnp.zeros_like(acc_ref)

    acc_ref[...] += jnp.dot(h_ref[...], w_ref[...],
                            preferred_element_type=jnp.float32)

    @pl.when(k == nk - 1)
    def _():
        o_ref[...] = _leaky(acc_ref[...] + b_ref[...], 0.01)


def _mlp2(h1, W2, b2):
    bn, bk = 1280, 640
    nk = HID // bk
    return pl.pallas_call(
        functools.partial(_mlp2_body, nk=nk),
        grid=(N // BM, HID // bn, nk),
        in_specs=[
            pl.BlockSpec((BM, bk), lambda m, n, k: (m, k)),
            pl.BlockSpec((bk, bn), lambda m, n, k: (k, n)),
            pl.BlockSpec((1, bn), lambda m, n, k: (0, n)),
        ],
        out_specs=pl.BlockSpec((BM, bn), lambda m, n, k: (m, n)),
        out_shape=jax.ShapeDtypeStruct((N, HID), jnp.float32),
        scratch_shapes=[pltpu.VMEM((BM, bn), jnp.float32)],
    )(h1, W2, b2.reshape(1, HID))


def _mlp34_body(h_ref, w3_ref, b3_ref, w4_ref, b4_ref, o_ref):
    h3 = jnp.dot(h_ref[...], w3_ref[...], preferred_element_type=jnp.float32)
    h3 = _leaky(h3 + b3_ref[...], 0.01)
    h4 = jnp.dot(h3, w4_ref[...], preferred_element_type=jnp.float32)
    o_ref[...] = jax.nn.sigmoid(h4 + b4_ref[...])


def _mlp34(h2, W3, b3, W4, b4):
    return pl.pallas_call(
        _mlp34_body,
        grid=(N // BM,),
        in_specs=[
            pl.BlockSpec((BM, HID), lambda m: (m, 0)),
            pl.BlockSpec((HID, IN_F), lambda m: (0, 0)),
            pl.BlockSpec((1, IN_F), lambda m: (0, 0)),
            pl.BlockSpec((IN_F, 1), lambda m: (0, 0)),
            pl.BlockSpec((1, 1), lambda m: (0, 0)),
        ],
        out_specs=pl.BlockSpec((BM, 1), lambda m: (m, 0)),
        out_shape=jax.ShapeDtypeStruct((N, 1), jnp.float32),
    )(h2, W3, b3.reshape(1, IN_F), W4, b4.reshape(1, 1))


def kernel(x, edge_index,
           att1_Wsrc, att1_bsrc, att1_Wdst, att1_bdst, att1_attn, att1_bias, att1_resW,
           def1_Wsrc, def1_bsrc, def1_Wdst, def1_bdst, def1_attn, def1_bias, def1_resW,
           def2_Wsrc, def2_bsrc, def2_Wdst, def2_bdst, def2_attn, def2_bias,
           W1, b1, W2, b2, W3, b3, W4, b4):
    srcb = edge_index[0].reshape(NW, NBLK, EB)
    dstb = edge_index[1].reshape(NW, NBLK, EB)

    Wcat = jnp.concatenate(
        [att1_Wsrc, def1_Wsrc, att1_Wdst, def1_Wdst], axis=1)
    bcat = jnp.concatenate(
        [att1_bsrc, def1_bsrc, att1_bdst, def1_bdst]).reshape(1, 128)
    fsrc1, fdst1 = _prep1(x, Wcat, bcat)
    attn1 = jnp.concatenate([att1_attn, def1_attn], axis=0)
    acc1 = _sc_edge_pass(4, fsrc1, fdst1, attn1, srcb, dstb)

    rWcat = jnp.concatenate([att1_resW, def1_resW], axis=1)
    biascat = jnp.concatenate([att1_bias, def1_bias]).reshape(1, 64)
    W2cat = jnp.concatenate([def2_Wsrc, def2_Wdst], axis=1)
    b2cat = jnp.concatenate([def2_bsrc, def2_bdst]).reshape(1, 64)
    hA, hD, f2s, f2d = _finish1(acc1, x, rWcat, biascat, W2cat, b2cat)

    acc2 = _sc_edge_pass(2, f2s, f2d, def2_attn, srcb, dstb)

    h1 = _finish2_mlp1(acc2, hA, hD, x, def2_bias.reshape(1, 32), W1, b1)
    h2 = _mlp2(h1, W2, b2)
    return _mlp34(h2, W3, b3, W4, b4)


# speedup vs baseline: 118.6970x; 1.1684x over previous
import functools

import jax
import jax.numpy as jnp
from jax import lax
from jax.experimental import pallas as pl
from jax.experimental.pallas import tpu as pltpu
from jax.experimental.pallas import tpu_sc as plsc

N = 10000
E = 640000
D = 16
H = 2
IN_F = 5 * D
HID = IN_F ** 2

NC, NS = 2, 16
NW = NC * NS
EB = 80
EPW = E // NW
NBLK = EPW // EB

BM = 1000


def _leaky(v, slope):
    return jnp.where(v > 0, v, slope * v)


def _elu(v):
    return jnp.where(v > 0, v, jnp.exp(v) - 1.0)


def _sc_edge_body(nh, fsrc_hbm, fdst_hbm, attn_hbm, srcb_hbm, dstb_hbm,
                  out_hbm, acc, attn_v, idxs_v, idxd_v, zb,
                  fs0, fd0, wb0, eb0, fs1, fd1, wb1, eb1,
                  sg0, ss0, sg1, ss1):
    fw = nh * 16
    aw = fw + 16
    c = lax.axis_index("c")
    s = lax.axis_index("s")
    w = s * NC + c
    zeros16 = jnp.zeros((16,), jnp.float32)

    for r in range(16):
        for j in range(aw // 16):
            zb[r, pl.ds(j * 16, 16)] = zeros16
    nz = jnp.where(s == NS - 1, 40, 39)

    def _zero_chunk(k, carry):
        off = pl.multiple_of(s * 624 + k * 16, 8)
        pltpu.sync_copy(zb, acc.at[pl.ds(off, 16)])
        return carry
    lax.fori_loop(0, nz, _zero_chunk, 0)

    def _wb_row(r, carry):
        wb0[r, pl.ds(fw, 16)] = zeros16
        wb1[r, pl.ds(fw, 16)] = zeros16
        return carry
    lax.fori_loop(0, EB, _wb_row, 0)

    pltpu.sync_copy(attn_hbm, attn_v)
    pltpu.sync_copy(srcb_hbm.at[w], idxs_v)
    pltpu.sync_copy(dstb_hbm.at[w], idxd_v)
    av = [attn_v[h, :] for h in range(nh)]

    plsc.subcore_barrier()

    slots = ((fs0, fd0, wb0, eb0, sg0, ss0), (fs1, fd1, wb1, eb1, sg1, ss1))

    def _fire_gathers(q, b):
        fs, fd, _, _, sg, _ = slots[b]
        pltpu.async_copy(fsrc_hbm.at[idxs_v.at[q]], fs, sg)
        pltpu.async_copy(fdst_hbm.at[idxd_v.at[q]], fd, sg)

    def _wait_gathers(q, b):
        fs, fd, _, _, sg, _ = slots[b]
        pltpu.make_async_copy(fsrc_hbm.at[idxs_v.at[q]], fs, sg).wait()
        pltpu.make_async_copy(fdst_hbm.at[idxd_v.at[q]], fd, sg).wait()

    iota16 = jnp.arange(16, dtype=jnp.int32)
    lane15 = iota16 == 15

    def _compute(q, b):
        fs, fd, wb, eb, _, _ = slots[b]

        @plsc.parallel_loop(0, EB, 1, unroll=4)
        def _edge_e(i):
            for h in range(nh):
                a = fs[i, pl.ds(16 * h, 16)]
                t = a + fd[i, pl.ds(16 * h, 16)]
                t = jnp.maximum(t, 0.2 * t)
                cs = plsc.cumsum(t * av[h])
                plsc.store_scatter(
                    eb,
                    [jnp.full((16,), h, jnp.int32),
                     jnp.full((16,), i, jnp.int32)],
                    cs, mask=lane15)

        for h in range(nh):
            for j in range(EB // 16):
                sl = pl.ds(j * 16, 16)
                ex = jnp.exp(eb[h, sl])
                eb[h, sl] = ex
                plsc.store_scatter(
                    wb,
                    [j * 16 + iota16, jnp.full((16,), fw + h, jnp.int32)],
                    ex)

        @plsc.parallel_loop(0, EB // 16, 1)
        def _group_w(g):
            base = g * 16
            for h in range(nh):
                ex16 = eb[h, pl.ds(base, 16)]
                for k in range(16):
                    i = base + k
                    wb[i, pl.ds(16 * h, 16)] = (
                        ex16[k] * fs[i, pl.ds(16 * h, 16)])

    def _fire_scatter(q, b):
        _, _, wb, _, _, ss = slots[b]
        pltpu.async_copy(wb, acc.at[idxd_v.at[q]], ss, add=True)

    def _wait_scatter(q, b):
        _, _, wb, _, _, ss = slots[b]
        pltpu.make_async_copy(wb, acc.at[idxd_v.at[q]], ss).wait()

    _fire_gathers(0, 0)
    _fire_gathers(1, 1)

    def _outer(g, carry):
        for b in range(2):
            q = 2 * g + b
            _wait_gathers(q, b)

            @pl.when(q >= 2)
            def _():
                _wait_scatter(q - 2, b)

            _compute(q, b)
            _fire_scatter(q, b)

            @pl.when(q + 2 < NBLK)
            def _():
                _fire_gathers(q + 2, b)
        return carry
    lax.fori_loop(0, NBLK // 2, _outer, 0)

    _wait_scatter(NBLK - 2, 0)
    _wait_scatter(NBLK - 1, 1)
    plsc.subcore_barrier()

    def _dump_chunk(k, carry):
        off = pl.multiple_of(s * 624 + k * 16, 8)
        pltpu.sync_copy(acc.at[pl.ds(off, 16)], out_hbm.at[c, pl.ds(off, 16)])
        return carry
    lax.fori_loop(0, nz, _dump_chunk, 0)


def _sc_edge_pass(nh, fsrc, fdst, attn_cat, srcb, dstb):
    fw = nh * 16
    aw = fw + 16
    mesh = plsc.VectorSubcoreMesh(core_axis_name="c", subcore_axis_name="s")
    kfn = pl.kernel(
        functools.partial(_sc_edge_body, nh),
        out_type=jax.ShapeDtypeStruct((NC, N, aw), jnp.float32),
        mesh=mesh,
        compiler_params=pltpu.CompilerParams(
            needs_layout_passes=False, use_tc_tiling_on_sc=False),
        scratch_types=[
            pltpu.VMEM_SHARED((N, aw), jnp.float32),
            pltpu.VMEM((nh, 16), jnp.float32),
            pltpu.VMEM((NBLK, EB), jnp.int32),
            pltpu.VMEM((NBLK, EB), jnp.int32),
            pltpu.VMEM((16, aw), jnp.float32),
            pltpu.VMEM((EB, fw), jnp.float32),
            pltpu.VMEM((EB, fw), jnp.float32),
            pltpu.VMEM((EB, aw), jnp.float32),
            pltpu.VMEM((nh, EB), jnp.float32),
            pltpu.VMEM((EB, fw), jnp.float32),
            pltpu.VMEM((EB, fw), jnp.float32),
            pltpu.VMEM((EB, aw), jnp.float32),
            pltpu.VMEM((nh, EB), jnp.float32),
            pltpu.SemaphoreType.DMA,
            pltpu.SemaphoreType.DMA,
            pltpu.SemaphoreType.DMA,
            pltpu.SemaphoreType.DMA,
        ],
    )
    return kfn(fsrc, fdst, attn_cat, srcb, dstb)


def _prep1_body(x_ref, w_ref, b_ref, os_ref, od_ref):
    f = jnp.dot(x_ref[...], w_ref[...], preferred_element_type=jnp.float32)
    f = f + b_ref[...]
    os_ref[...] = f[:, :64]
    od_ref[...] = f[:, 64:]


def _prep1(x, Wcat, bcat):
    return pl.pallas_call(
        _prep1_body,
        grid=(N // BM,),
        in_specs=[
            pl.BlockSpec((BM, D), lambda m: (m, 0)),
            pl.BlockSpec((D, 128), lambda m: (0, 0)),
            pl.BlockSpec((1, 128), lambda m: (0, 0)),
        ],
        out_specs=[
            pl.BlockSpec((BM, 64), lambda m: (m, 0)),
            pl.BlockSpec((BM, 64), lambda m: (m, 0)),
        ],
        out_shape=[
            jax.ShapeDtypeStruct((N, 64), jnp.float32),
            jax.ShapeDtypeStruct((N, 64), jnp.float32),
        ],
    )(x, Wcat, bcat)


def _finish1_body(acc_ref, x_ref, rw_ref, bc_ref, w2c_ref, b2c_ref,
                  ha_ref, hd_ref, f2s_ref, f2d_ref):
    a = acc_ref[0] + acc_ref[1]
    res = jnp.dot(x_ref[...], rw_ref[...], preferred_element_type=jnp.float32)
    parts = []
    for h in range(4):
        den = jnp.maximum(a[:, 64 + h:65 + h], 1e-16)
        parts.append(a[:, 16 * h:16 * h + 16] / den)
    rst = jnp.concatenate(parts, axis=1) + res + bc_ref[...]
    hcat = _elu(rst)
    ha_ref[...] = hcat[:, :32]
    hd = hcat[:, 32:]
    hd_ref[...] = hd
    f2 = jnp.dot(hd, w2c_ref[...], preferred_element_type=jnp.float32)
    f2 = f2 + b2c_ref[...]
    f2s_ref[...] = f2[:, :32]
    f2d_ref[...] = f2[:, 32:]


def _finish1(acc1, x, rWcat, biascat, W2cat, b2cat):
    return pl.pallas_call(
        _finish1_body,
        grid=(N // BM,),
        in_specs=[
            pl.BlockSpec((NC, BM, 80), lambda m: (0, m, 0)),
            pl.BlockSpec((BM, D), lambda m: (m, 0)),
            pl.BlockSpec((D, 64), lambda m: (0, 0)),
            pl.BlockSpec((1, 64), lambda m: (0, 0)),
            pl.BlockSpec((32, 64), lambda m: (0, 0)),
            pl.BlockSpec((1, 64), lambda m: (0, 0)),
        ],
        out_specs=[
            pl.BlockSpec((BM, 32), lambda m: (m, 0)),
            pl.BlockSpec((BM, 32), lambda m: (m, 0)),
            pl.BlockSpec((BM, 32), lambda m: (m, 0)),
            pl.BlockSpec((BM, 32), lambda m: (m, 0)),
        ],
        out_shape=[jax.ShapeDtypeStruct((N, 32), jnp.float32)] * 4,
    )(acc1, x, rWcat, biascat, W2cat, b2cat)


def _finish2_body(acc_ref, ha_ref, hd_ref, x_ref, b2_ref, w1_ref, b1_ref,
                  o_ref):
    a = acc_ref[0] + acc_ref[1]
    parts = []
    for h in range(2):
        den = jnp.maximum(a[:, 32 + h:33 + h], 1e-16)
        parts.append(a[:, 16 * h:16 * h + 16] / den)
    rst = jnp.concatenate(parts, axis=1) + hd_ref[...] + b2_ref[...]
    hd2 = _elu(rst)
    z = jnp.concatenate([ha_ref[...], hd2, x_ref[...]], axis=1)
    acc = jnp.dot(z, w1_ref[...], preferred_element_type=jnp.float32)
    o_ref[...] = _leaky(acc + b1_ref[...], 0.01).astype(jnp.bfloat16)


def _finish2_mlp1(acc2, hA, hD, x, bias2, W1, b1):
    return pl.pallas_call(
        _finish2_body,
        grid=(N // BM,),
        in_specs=[
            pl.BlockSpec((NC, BM, 48), lambda m: (0, m, 0)),
            pl.BlockSpec((BM, 32), lambda m: (m, 0)),
            pl.BlockSpec((BM, 32), lambda m: (m, 0)),
            pl.BlockSpec((BM, D), lambda m: (m, 0)),
            pl.BlockSpec((1, 32), lambda m: (0, 0)),
            pl.BlockSpec((IN_F, HID), lambda m: (0, 0)),
            pl.BlockSpec((1, HID), lambda m: (0, 0)),
        ],
        out_specs=pl.BlockSpec((BM, HID), lambda m: (m, 0)),
        out_shape=jax.ShapeDtypeStruct((N, HID), jnp.bfloat16),
    )(acc2, hA, hD, x, bias2, W1, b1.reshape(1, HID))


def _mlp2_body(h_ref, w_ref, b_ref, o_ref, acc_ref, *, nk):
    k = pl.program_id(2)
    part = jnp.dot(h_ref[...], w_ref[...], preferred_element_type=jnp.float32)

    @pl.when(k == 0)
    def _():
        acc_ref[...] = part

    @pl.when(k > 0)
    def _():
        acc_ref[...] += part

    @pl.when(k == nk - 1)
    def _():
        o_ref[...] = _leaky(acc_ref[...] + b_ref[...],
                            0.01).astype(jnp.bfloat16)


def _mlp2(h1, W2, b2):
    bm, bn, bk = 2000, 1280, 1280
    nk = HID // bk
    return pl.pallas_call(
        functools.partial(_mlp2_body, nk=nk),
        grid=(N // bm, HID // bn, nk),
        in_specs=[
            pl.BlockSpec((bm, bk), lambda m, n, k: (m, k)),
            pl.BlockSpec((bk, bn), lambda m, n, k: (k, n)),
            pl.BlockSpec((1, bn), lambda m, n, k: (0, n)),
        ],
        out_specs=pl.BlockSpec((bm, bn), lambda m, n, k: (m, n)),
        out_shape=jax.ShapeDtypeStruct((N, HID), jnp.bfloat16),
        scratch_shapes=[pltpu.VMEM((bm, bn), jnp.float32)],
    )(h1, W2.astype(jnp.bfloat16), b2.reshape(1, HID))


def _mlp34_body(h_ref, w3_ref, b3_ref, w4_ref, b4_ref, o_ref):
    h3 = jnp.dot(h_ref[...], w3_ref[...].astype(jnp.bfloat16),
                 preferred_element_type=jnp.float32)
    h3 = _leaky(h3 + b3_ref[...], 0.01)
    h4 = jnp.dot(h3, w4_ref[...], preferred_element_type=jnp.float32)
    o_ref[...] = jax.nn.sigmoid(h4 + b4_ref[...])


def _mlp34(h2, W3, b3, W4, b4):
    return pl.pallas_call(
        _mlp34_body,
        grid=(N // BM,),
        in_specs=[
            pl.BlockSpec((BM, HID), lambda m: (m, 0)),
            pl.BlockSpec((HID, IN_F), lambda m: (0, 0)),
            pl.BlockSpec((1, IN_F), lambda m: (0, 0)),
            pl.BlockSpec((IN_F, 1), lambda m: (0, 0)),
            pl.BlockSpec((1, 1), lambda m: (0, 0)),
        ],
        out_specs=pl.BlockSpec((BM, 1), lambda m: (m, 0)),
        out_shape=jax.ShapeDtypeStruct((N, 1), jnp.float32),
    )(h2, W3, b3.reshape(1, IN_F), W4, b4.reshape(1, 1))


def kernel(x, edge_index,
           att1_Wsrc, att1_bsrc, att1_Wdst, att1_bdst, att1_attn, att1_bias, att1_resW,
           def1_Wsrc, def1_bsrc, def1_Wdst, def1_bdst, def1_attn, def1_bias, def1_resW,
           def2_Wsrc, def2_bsrc, def2_Wdst, def2_bdst, def2_attn, def2_bias,
           W1, b1, W2, b2, W3, b3, W4, b4):
    srcb = edge_index[0].reshape(NW, NBLK, EB)
    dstb = edge_index[1].reshape(NW, NBLK, EB)

    Wcat = jnp.concatenate(
        [att1_Wsrc, def1_Wsrc, att1_Wdst, def1_Wdst], axis=1)
    bcat = jnp.concatenate(
        [att1_bsrc, def1_bsrc, att1_bdst, def1_bdst]).reshape(1, 128)
    fsrc1, fdst1 = _prep1(x, Wcat, bcat)
    attn1 = jnp.concatenate([att1_attn, def1_attn], axis=0)
    acc1 = _sc_edge_pass(4, fsrc1, fdst1, attn1, srcb, dstb)

    rWcat = jnp.concatenate([att1_resW, def1_resW], axis=1)
    biascat = jnp.concatenate([att1_bias, def1_bias]).reshape(1, 64)
    W2cat = jnp.concatenate([def2_Wsrc, def2_Wdst], axis=1)
    b2cat = jnp.concatenate([def2_bsrc, def2_bdst]).reshape(1, 64)
    hA, hD, f2s, f2d = _finish1(acc1, x, rWcat, biascat, W2cat, b2cat)

    acc2 = _sc_edge_pass(2, f2s, f2d, def2_attn, srcb, dstb)

    h1 = _finish2_mlp1(acc2, hA, hD, x, def2_bias.reshape(1, 32), W1, b1)
    h2 = _mlp2(h1, W2, b2)
    return _mlp34(h2, W3, b3, W4, b4)


# speedup vs baseline: 120.9121x; 1.0187x over previous
import functools

import jax
import jax.numpy as jnp
from jax import lax
from jax.experimental import pallas as pl
from jax.experimental.pallas import tpu as pltpu
from jax.experimental.pallas import tpu_sc as plsc

N = 10000
E = 640000
D = 16
H = 2
IN_F = 5 * D
HID = IN_F ** 2

NC, NS = 2, 16
NW = NC * NS
EB = 80
EPW = E // NW
NBLK = EPW // EB

BM = 1000


def _leaky(v, slope):
    return jnp.where(v > 0, v, slope * v)


def _elu(v):
    return jnp.where(v > 0, v, jnp.exp(v) - 1.0)


def _sc_edge_body(nh, fsrc_hbm, fdst_hbm, attn_hbm, srcb_hbm, dstb_hbm,
                  out_hbm, acc, attn_v, idxs_v, idxd_v, zb,
                  fs0, fd0, wb0, eb0, fs1, fd1, wb1, eb1,
                  sg0, ss0, sg1, ss1):
    fw = nh * 16
    aw = fw + 16
    c = lax.axis_index("c")
    s = lax.axis_index("s")
    w = s * NC + c
    zeros16 = jnp.zeros((16,), jnp.float32)

    for r in range(16):
        for j in range(aw // 16):
            zb[r, pl.ds(j * 16, 16)] = zeros16
    nz = jnp.where(s == NS - 1, 40, 39)

    def _zero_chunk(k, carry):
        off = pl.multiple_of(s * 624 + k * 16, 8)
        pltpu.sync_copy(zb, acc.at[pl.ds(off, 16)])
        return carry
    lax.fori_loop(0, nz, _zero_chunk, 0)

    def _wb_row(r, carry):
        wb0[r, pl.ds(fw, 16)] = zeros16
        wb1[r, pl.ds(fw, 16)] = zeros16
        return carry
    lax.fori_loop(0, EB, _wb_row, 0)

    pltpu.sync_copy(attn_hbm, attn_v)
    pltpu.sync_copy(srcb_hbm.at[w], idxs_v)
    pltpu.sync_copy(dstb_hbm.at[w], idxd_v)
    av = [attn_v[h, :] for h in range(nh)]

    plsc.subcore_barrier()

    slots = ((fs0, fd0, wb0, eb0, sg0, ss0), (fs1, fd1, wb1, eb1, sg1, ss1))

    def _fire_gathers(q, b):
        fs, fd, _, _, sg, _ = slots[b]
        pltpu.async_copy(fsrc_hbm.at[idxs_v.at[q]], fs, sg)
        pltpu.async_copy(fdst_hbm.at[idxd_v.at[q]], fd, sg)

    def _wait_gathers(q, b):
        fs, fd, _, _, sg, _ = slots[b]
        pltpu.make_async_copy(fsrc_hbm.at[idxs_v.at[q]], fs, sg).wait()
        pltpu.make_async_copy(fdst_hbm.at[idxd_v.at[q]], fd, sg).wait()

    iota16 = jnp.arange(16, dtype=jnp.int32)
    lane15 = iota16 == 15

    def _compute(q, b):
        fs, fd, wb, eb, _, _ = slots[b]

        @plsc.parallel_loop(0, EB, 1, unroll=8)
        def _edge_e(i):
            for h in range(nh):
                a = fs[i, pl.ds(16 * h, 16)]
                t = a + fd[i, pl.ds(16 * h, 16)]
                t = jnp.maximum(t, 0.2 * t)
                cs = plsc.cumsum(t * av[h])
                plsc.store_scatter(
                    eb,
                    [jnp.full((16,), h, jnp.int32),
                     jnp.full((16,), i, jnp.int32)],
                    cs, mask=lane15)

        for h in range(nh):
            for j in range(EB // 16):
                sl = pl.ds(j * 16, 16)
                ex = jnp.exp(eb[h, sl])
                eb[h, sl] = ex
                plsc.store_scatter(
                    wb,
                    [j * 16 + iota16, jnp.full((16,), fw + h, jnp.int32)],
                    ex)

        @plsc.parallel_loop(0, EB // 16, 1, unroll=2)
        def _group_w(g):
            base = g * 16
            for h in range(nh):
                ex16 = eb[h, pl.ds(base, 16)]
                for k in range(16):
                    i = base + k
                    wb[i, pl.ds(16 * h, 16)] = (
                        ex16[k] * fs[i, pl.ds(16 * h, 16)])

    def _fire_scatter(q, b):
        _, _, wb, _, _, ss = slots[b]
        pltpu.async_copy(wb, acc.at[idxd_v.at[q]], ss, add=True)

    def _wait_scatter(q, b):
        _, _, wb, _, _, ss = slots[b]
        pltpu.make_async_copy(wb, acc.at[idxd_v.at[q]], ss).wait()

    _fire_gathers(0, 0)
    _fire_gathers(1, 1)

    def _outer(g, carry):
        for b in range(2):
            q = 2 * g + b
            _wait_gathers(q, b)

            @pl.when(q >= 2)
            def _():
                _wait_scatter(q - 2, b)

            _compute(q, b)
            _fire_scatter(q, b)

            @pl.when(q + 2 < NBLK)
            def _():
                _fire_gathers(q + 2, b)
        return carry
    lax.fori_loop(0, NBLK // 2, _outer, 0)

    _wait_scatter(NBLK - 2, 0)
    _wait_scatter(NBLK - 1, 1)
    plsc.subcore_barrier()

    def _dump_chunk(k, carry):
        off = pl.multiple_of(s * 624 + k * 16, 8)
        pltpu.sync_copy(acc.at[pl.ds(off, 16)], out_hbm.at[c, pl.ds(off, 16)])
        return carry
    lax.fori_loop(0, nz, _dump_chunk, 0)


def _sc_edge_pass(nh, fsrc, fdst, attn_cat, srcb, dstb):
    fw = nh * 16
    aw = fw + 16
    mesh = plsc.VectorSubcoreMesh(core_axis_name="c", subcore_axis_name="s")
    kfn = pl.kernel(
        functools.partial(_sc_edge_body, nh),
        out_type=jax.ShapeDtypeStruct((NC, N, aw), jnp.float32),
        mesh=mesh,
        compiler_params=pltpu.CompilerParams(
            needs_layout_passes=False, use_tc_tiling_on_sc=False),
        scratch_types=[
            pltpu.VMEM_SHARED((N, aw), jnp.float32),
            pltpu.VMEM((nh, 16), jnp.float32),
            pltpu.VMEM((NBLK, EB), jnp.int32),
            pltpu.VMEM((NBLK, EB), jnp.int32),
            pltpu.VMEM((16, aw), jnp.float32),
            pltpu.VMEM((EB, fw), jnp.float32),
            pltpu.VMEM((EB, fw), jnp.float32),
            pltpu.VMEM((EB, aw), jnp.float32),
            pltpu.VMEM((nh, EB), jnp.float32),
            pltpu.VMEM((EB, fw), jnp.float32),
            pltpu.VMEM((EB, fw), jnp.float32),
            pltpu.VMEM((EB, aw), jnp.float32),
            pltpu.VMEM((nh, EB), jnp.float32),
            pltpu.SemaphoreType.DMA,
            pltpu.SemaphoreType.DMA,
            pltpu.SemaphoreType.DMA,
            pltpu.SemaphoreType.DMA,
        ],
    )
    return kfn(fsrc, fdst, attn_cat, srcb, dstb)


def _prep1_body(x_ref, w_ref, b_ref, os_ref, od_ref):
    f = jnp.dot(x_ref[...], w_ref[...], preferred_element_type=jnp.float32)
    f = f + b_ref[...]
    os_ref[...] = f[:, :64]
    od_ref[...] = f[:, 64:]


def _prep1(x, Wcat, bcat):
    return pl.pallas_call(
        _prep1_body,
        grid=(N // BM,),
        in_specs=[
            pl.BlockSpec((BM, D), lambda m: (m, 0)),
            pl.BlockSpec((D, 128), lambda m: (0, 0)),
            pl.BlockSpec((1, 128), lambda m: (0, 0)),
        ],
        out_specs=[
            pl.BlockSpec((BM, 64), lambda m: (m, 0)),
            pl.BlockSpec((BM, 64), lambda m: (m, 0)),
        ],
        out_shape=[
            jax.ShapeDtypeStruct((N, 64), jnp.float32),
            jax.ShapeDtypeStruct((N, 64), jnp.float32),
        ],
    )(x, Wcat, bcat)


def _finish1_body(acc_ref, x_ref, rw_ref, bc_ref, w2c_ref, b2c_ref,
                  ha_ref, hd_ref, f2s_ref, f2d_ref):
    a = acc_ref[0] + acc_ref[1]
    res = jnp.dot(x_ref[...], rw_ref[...], preferred_element_type=jnp.float32)
    parts = []
    for h in range(4):
        den = jnp.maximum(a[:, 64 + h:65 + h], 1e-16)
        parts.append(a[:, 16 * h:16 * h + 16] / den)
    rst = jnp.concatenate(parts, axis=1) + res + bc_ref[...]
    hcat = _elu(rst)
    ha_ref[...] = hcat[:, :32]
    hd = hcat[:, 32:]
    hd_ref[...] = hd
    f2 = jnp.dot(hd, w2c_ref[...], preferred_element_type=jnp.float32)
    f2 = f2 + b2c_ref[...]
    f2s_ref[...] = f2[:, :32]
    f2d_ref[...] = f2[:, 32:]


def _finish1(acc1, x, rWcat, biascat, W2cat, b2cat):
    return pl.pallas_call(
        _finish1_body,
        grid=(N // BM,),
        in_specs=[
            pl.BlockSpec((NC, BM, 80), lambda m: (0, m, 0)),
            pl.BlockSpec((BM, D), lambda m: (m, 0)),
            pl.BlockSpec((D, 64), lambda m: (0, 0)),
            pl.BlockSpec((1, 64), lambda m: (0, 0)),
            pl.BlockSpec((32, 64), lambda m: (0, 0)),
            pl.BlockSpec((1, 64), lambda m: (0, 0)),
        ],
        out_specs=[
            pl.BlockSpec((BM, 32), lambda m: (m, 0)),
            pl.BlockSpec((BM, 32), lambda m: (m, 0)),
            pl.BlockSpec((BM, 32), lambda m: (m, 0)),
            pl.BlockSpec((BM, 32), lambda m: (m, 0)),
        ],
        out_shape=[jax.ShapeDtypeStruct((N, 32), jnp.float32)] * 4,
    )(acc1, x, rWcat, biascat, W2cat, b2cat)


def _finish2_body(acc_ref, ha_ref, hd_ref, x_ref, b2_ref, w1_ref, b1_ref,
                  o_ref):
    a = acc_ref[0] + acc_ref[1]
    parts = []
    for h in range(2):
        den = jnp.maximum(a[:, 32 + h:33 + h], 1e-16)
        parts.append(a[:, 16 * h:16 * h + 16] / den)
    rst = jnp.concatenate(parts, axis=1) + hd_ref[...] + b2_ref[...]
    hd2 = _elu(rst)
    z = jnp.concatenate([ha_ref[...], hd2, x_ref[...]], axis=1)
    acc = jnp.dot(z, w1_ref[...], preferred_element_type=jnp.float32)
    o_ref[...] = _leaky(acc + b1_ref[...], 0.01).astype(jnp.bfloat16)


def _finish2_mlp1(acc2, hA, hD, x, bias2, W1, b1):
    return pl.pallas_call(
        _finish2_body,
        grid=(N // BM,),
        in_specs=[
            pl.BlockSpec((NC, BM, 48), lambda m: (0, m, 0)),
            pl.BlockSpec((BM, 32), lambda m: (m, 0)),
            pl.BlockSpec((BM, 32), lambda m: (m, 0)),
            pl.BlockSpec((BM, D), lambda m: (m, 0)),
            pl.BlockSpec((1, 32), lambda m: (0, 0)),
            pl.BlockSpec((IN_F, HID), lambda m: (0, 0)),
            pl.BlockSpec((1, HID), lambda m: (0, 0)),
        ],
        out_specs=pl.BlockSpec((BM, HID), lambda m: (m, 0)),
        out_shape=jax.ShapeDtypeStruct((N, HID), jnp.bfloat16),
    )(acc2, hA, hD, x, bias2, W1, b1.reshape(1, HID))


def _mlp2_body(h_ref, w_ref, b_ref, o_ref, acc_ref, *, nk):
    k = pl.program_id(2)
    part = jnp.dot(h_ref[...], w_ref[...], preferred_element_type=jnp.float32)

    @pl.when(k == 0)
    def _():
        acc_ref[...] = part

    @pl.when(k > 0)
    def _():
        acc_ref[...] += part

    @pl.when(k == nk - 1)
    def _():
        o_ref[...] = _leaky(acc_ref[...] + b_ref[...],
                            0.01).astype(jnp.bfloat16)


def _mlp2(h1, W2, b2):
    bm, bn, bk = 2000, 1280, 1280
    nk = HID // bk
    return pl.pallas_call(
        functools.partial(_mlp2_body, nk=nk),
        grid=(N // bm, HID // bn, nk),
        in_specs=[
            pl.BlockSpec((bm, bk), lambda m, n, k: (m, k)),
            pl.BlockSpec((bk, bn), lambda m, n, k: (k, n)),
            pl.BlockSpec((1, bn), lambda m, n, k: (0, n)),
        ],
        out_specs=pl.BlockSpec((bm, bn), lambda m, n, k: (m, n)),
        out_shape=jax.ShapeDtypeStruct((N, HID), jnp.bfloat16),
        scratch_shapes=[pltpu.VMEM((bm, bn), jnp.float32)],
    )(h1, W2.astype(jnp.bfloat16), b2.reshape(1, HID))


def _mlp34_body(h_ref, w3_ref, b3_ref, w4_ref, b4_ref, o_ref):
    h3 = jnp.dot(h_ref[...], w3_ref[...].astype(jnp.bfloat16),
                 preferred_element_type=jnp.float32)
    h3 = _leaky(h3 + b3_ref[...], 0.01)
    h4 = jnp.dot(h3, w4_ref[...], preferred_element_type=jnp.float32)
    o_ref[...] = jax.nn.sigmoid(h4 + b4_ref[...])


def _mlp34(h2, W3, b3, W4, b4):
    return pl.pallas_call(
        _mlp34_body,
        grid=(N // BM,),
        in_specs=[
            pl.BlockSpec((BM, HID), lambda m: (m, 0)),
            pl.BlockSpec((HID, IN_F), lambda m: (0, 0)),
            pl.BlockSpec((1, IN_F), lambda m: (0, 0)),
            pl.BlockSpec((IN_F, 1), lambda m: (0, 0)),
            pl.BlockSpec((1, 1), lambda m: (0, 0)),
        ],
        out_specs=pl.BlockSpec((BM, 1), lambda m: (m, 0)),
        out_shape=jax.ShapeDtypeStruct((N, 1), jnp.float32),
    )(h2, W3, b3.reshape(1, IN_F), W4, b4.reshape(1, 1))


def kernel(x, edge_index,
           att1_Wsrc, att1_bsrc, att1_Wdst, att1_bdst, att1_attn, att1_bias, att1_resW,
           def1_Wsrc, def1_bsrc, def1_Wdst, def1_bdst, def1_attn, def1_bias, def1_resW,
           def2_Wsrc, def2_bsrc, def2_Wdst, def2_bdst, def2_attn, def2_bias,
           W1, b1, W2, b2, W3, b3, W4, b4):
    srcb = edge_index[0].reshape(NW, NBLK, EB)
    dstb = edge_index[1].reshape(NW, NBLK, EB)

    Wcat = jnp.concatenate(
        [att1_Wsrc, def1_Wsrc, att1_Wdst, def1_Wdst], axis=1)
    bcat = jnp.concatenate(
        [att1_bsrc, def1_bsrc, att1_bdst, def1_bdst]).reshape(1, 128)
    fsrc1, fdst1 = _prep1(x, Wcat, bcat)
    attn1 = jnp.concatenate([att1_attn, def1_attn], axis=0)
    acc1 = _sc_edge_pass(4, fsrc1, fdst1, attn1, srcb, dstb)

    rWcat = jnp.concatenate([att1_resW, def1_resW], axis=1)
    biascat = jnp.concatenate([att1_bias, def1_bias]).reshape(1, 64)
    W2cat = jnp.concatenate([def2_Wsrc, def2_Wdst], axis=1)
    b2cat = jnp.concatenate([def2_bsrc, def2_bdst]).reshape(1, 64)
    hA, hD, f2s, f2d = _finish1(acc1, x, rWcat, biascat, W2cat, b2cat)

    acc2 = _sc_edge_pass(2, f2s, f2d, def2_attn, srcb, dstb)

    h1 = _finish2_mlp1(acc2, hA, hD, x, def2_bias.reshape(1, 32), W1, b1)
    h2 = _mlp2(h1, W2, b2)
    return _mlp34(h2, W3, b3, W4, b4)
